# Initial kernel scaffold; baseline (speedup 1.0000x reference)
import jax
import jax.numpy as jnp
from jax.experimental import pallas as pl


def kernel(trg_token_ids_batch):
    raise NotImplementedError("write your pallas kernel here")


# speedup vs baseline: 1.8888x; 1.8888x over previous
import jax
import jax.numpy as jnp
from jax.experimental import pallas as pl

_V = 100000
_B = 1024
_SMOOTH = 0.1
_CONF = 1.0 - _SMOOTH
_FILL = _SMOOTH / (_V - 2)
_VB = 2048


def _fill_block(trg_ref, out_ref):
    j = pl.program_id(0)
    t = trg_ref[...]
    col = jax.lax.broadcasted_iota(jnp.int32, (_B, _VB), 1) + j * _VB
    hit = col == t
    zero = (col == 0) | (t == 0)
    val = jnp.where(hit, _CONF, _FILL)
    out_ref[...] = jnp.where(zero, 0.0, val)


def kernel(trg_token_ids_batch):
    grid = (_V + _VB - 1) // _VB
    return pl.pallas_call(
        _fill_block,
        grid=(grid,),
        in_specs=[pl.BlockSpec((_B, 1), lambda j: (0, 0))],
        out_specs=pl.BlockSpec((_B, _VB), lambda j: (0, j)),
        out_shape=jax.ShapeDtypeStruct((_B, _V), jnp.float32),
    )(trg_token_ids_batch)


# speedup vs baseline: 1.8971x; 1.0044x over previous
import jax
import jax.numpy as jnp
from jax.experimental import pallas as pl

_V = 100000
_B = 1024
_SMOOTH = 0.1
_CONF = 1.0 - _SMOOTH
_FILL = _SMOOTH / (_V - 2)
_VB = 2048


def _fill_block(trg_ref, out_ref):
    j = pl.program_id(0)
    t = trg_ref[...]
    col = jax.lax.broadcasted_iota(jnp.int32, (_B, _VB), 1) + j * _VB
    base = jnp.where(t == 0, 0.0, _FILL)
    val = jnp.where(col == t, _CONF, base)
    @pl.when(j == 0)
    def _():
        out_ref[...] = jnp.where(col == 0, 0.0, val)

    @pl.when(j != 0)
    def _():
        out_ref[...] = val


def kernel(trg_token_ids_batch):
    grid = (_V + _VB - 1) // _VB
    return pl.pallas_call(
        _fill_block,
        grid=(grid,),
        in_specs=[pl.BlockSpec((_B, 1), lambda j: (0, 0))],
        out_specs=pl.BlockSpec((_B, _VB), lambda j: (0, j)),
        out_shape=jax.ShapeDtypeStruct((_B, _V), jnp.float32),
    )(trg_token_ids_batch)
